# Initial kernel scaffold; baseline (speedup 1.0000x reference)
import jax
import jax.numpy as jnp
from jax.experimental import pallas as pl


def kernel(x, edge_index, W_lin, b_lin, W_root, b_root):
    raise NotImplementedError("write your pallas kernel here")


# speedup vs baseline: 4.4535x; 4.4535x over previous
import functools

import jax
import jax.numpy as jnp
from jax import lax
from jax.experimental import pallas as pl
from jax.experimental.pallas import tpu as pltpu
from jax.experimental.pallas import tpu_sc as plsc

N_NODES = 10000
D = 128
N_EDGES = 320000

NC = 2
NS = 16
NW = NC * NS

CHUNK = 128
ACC_ROWS = 10240
ROWS_PER_TILE = ACC_ROWS // NS
CHUNKS_PER_W = -(-N_EDGES // (NW * CHUNK))
EDGES_PAD = NW * CHUNK * CHUNKS_PER_W
DUMP_ROW = N_NODES


def _sc_segment_sum(x, src_w, dst_w, zrows, zdeg):
    mesh = plsc.VectorSubcoreMesh(
        core_axis_name="c", subcore_axis_name="s", num_cores=NC, num_subcores=NS
    )

    @functools.partial(
        pl.kernel,
        mesh=mesh,
        out_type=(
            jax.ShapeDtypeStruct((NC, ACC_ROWS, D), jnp.float32),
            jax.ShapeDtypeStruct((NC, ACC_ROWS), jnp.float32),
        ),
        scratch_types=[
            pltpu.VMEM_SHARED((ACC_ROWS, D), jnp.float32),
            pltpu.VMEM_SHARED((ACC_ROWS,), jnp.float32),
            pltpu.VMEM((CHUNKS_PER_W, CHUNK), jnp.int32),
            pltpu.VMEM((CHUNKS_PER_W, CHUNK), jnp.int32),
            pltpu.VMEM((CHUNK, D), jnp.float32),
            pltpu.VMEM((CHUNK,), jnp.float32),
        ],
    )
    def seg_sum(x_hbm, src_hbm, dst_hbm, zr_hbm, zd_hbm, part_hbm, deg_hbm,
                acc, dacc, srcv, dstv, rows, ones):
        cid = lax.axis_index("c")
        sid = lax.axis_index("s")
        wid = sid * NC + cid
        base = sid * ROWS_PER_TILE

        pltpu.sync_copy(zr_hbm.at[pl.ds(base, ROWS_PER_TILE)],
                        acc.at[pl.ds(base, ROWS_PER_TILE)])
        pltpu.sync_copy(zd_hbm.at[pl.ds(base, ROWS_PER_TILE)],
                        dacc.at[pl.ds(base, ROWS_PER_TILE)])
        for i in range(CHUNK // 16):
            ones[pl.ds(i * 16, 16)] = jnp.full((16,), 1.0, jnp.float32)

        pltpu.sync_copy(src_hbm.at[wid], srcv)
        pltpu.sync_copy(dst_hbm.at[wid], dstv)
        plsc.subcore_barrier()

        def body(j, _):
            pltpu.sync_copy(x_hbm.at[srcv.at[j]], rows)
            pltpu.sync_copy(rows, acc.at[dstv.at[j]], add=True)
            pltpu.sync_copy(ones, dacc.at[dstv.at[j]], add=True)
            return ()

        lax.fori_loop(0, CHUNKS_PER_W, body, ())
        plsc.subcore_barrier()

        pltpu.sync_copy(acc.at[pl.ds(base, ROWS_PER_TILE)],
                        part_hbm.at[cid].at[pl.ds(base, ROWS_PER_TILE)])
        pltpu.sync_copy(dacc.at[pl.ds(base, ROWS_PER_TILE)],
                        deg_hbm.at[cid].at[pl.ds(base, ROWS_PER_TILE)])

    return seg_sum(x, src_w, dst_w, zrows, zdeg)


ROW_BLK = 1000


def _tc_body(p0_ref, p1_ref, x_ref, wl_ref, wr_ref, bl_ref, br_ref, d0_ref,
             d1_ref, o_ref):
    agg = p0_ref[...] + p1_ref[...]
    acc = jnp.dot(agg, wl_ref[...], preferred_element_type=jnp.float32)
    acc += jnp.dot(x_ref[...], wr_ref[...], preferred_element_type=jnp.float32)
    acc += (d0_ref[...] + d1_ref[...]) * bl_ref[...]
    acc += br_ref[...]
    o_ref[...] = jnp.maximum(acc, 0.0)


def _tc_epilogue(p0, p1, x, wl_t, wr_t, bl, br, d0, d1):
    grid = (N_NODES // ROW_BLK,)
    blk = lambda i: (i, 0)
    full = lambda i: (0, 0)
    return pl.pallas_call(
        _tc_body,
        grid=grid,
        in_specs=[
            pl.BlockSpec((ROW_BLK, D), blk),
            pl.BlockSpec((ROW_BLK, D), blk),
            pl.BlockSpec((ROW_BLK, D), blk),
            pl.BlockSpec((D, D), full),
            pl.BlockSpec((D, D), full),
            pl.BlockSpec((1, D), full),
            pl.BlockSpec((1, D), full),
            pl.BlockSpec((ROW_BLK, 1), blk),
            pl.BlockSpec((ROW_BLK, 1), blk),
        ],
        out_specs=pl.BlockSpec((ROW_BLK, D), blk),
        out_shape=jax.ShapeDtypeStruct((N_NODES, D), jnp.float32),
    )(p0, p1, x, wl_t, wr_t, bl, br, d0, d1)


@jax.jit
def kernel(x, edge_index, W_lin, b_lin, W_root, b_root):
    src = edge_index[0].astype(jnp.int32)
    dst = edge_index[1].astype(jnp.int32)
    pad = EDGES_PAD - N_EDGES
    src_w = jnp.concatenate([src, jnp.zeros((pad,), jnp.int32)])
    dst_w = jnp.concatenate([dst, jnp.full((pad,), DUMP_ROW, jnp.int32)])
    src_w = src_w.reshape(NW, CHUNKS_PER_W, CHUNK)
    dst_w = dst_w.reshape(NW, CHUNKS_PER_W, CHUNK)
    zrows = jnp.zeros((ACC_ROWS, D), jnp.float32)
    zdeg = jnp.zeros((ACC_ROWS,), jnp.float32)

    part, degp = _sc_segment_sum(x, src_w, dst_w, zrows, zdeg)

    out = _tc_epilogue(
        part[0, :N_NODES], part[1, :N_NODES], x,
        W_lin.T, W_root.T,
        b_lin.reshape(1, D), b_root.reshape(1, D),
        degp[0, :N_NODES].reshape(N_NODES, 1),
        degp[1, :N_NODES].reshape(N_NODES, 1),
    )
    return out
